# baseline (speedup 1.0000x reference)
import jax
import jax.numpy as jnp
from jax import lax
from jax.experimental import pallas as pl
from jax.experimental.pallas import tpu as pltpu
from jax.experimental.pallas import tpu_sc as plsc

NC = 2
NS = 16
NW = NC * NS
L = 16
B = 16384
D = 64
BPW = B // NW
C = 8
NCH = BPW // C
NSL = D // L


def _sc_body(h_hbm, r_hbm, t_hbm, e1_hbm, e2_hbm, r1_hbm, r2_hbm, out_hbm,
             hidx_v, ridx_v, tidx_v,
             e1h0, e2h0, r10, r20, e1t0, e2t0,
             e1h1, e2h1, r11, r21, e1t1, e2t1,
             out_v, sem0, sem1):
    cid = lax.axis_index("c")
    sid = lax.axis_index("s")
    wid = sid * NC + cid
    base = wid * BPW
    lane = lax.iota(jnp.int32, L)

    pltpu.sync_copy(h_hbm.at[pl.ds(base, BPW)], hidx_v.at[pl.ds(0, BPW)])
    pltpu.sync_copy(r_hbm.at[pl.ds(base, BPW)], ridx_v.at[pl.ds(0, BPW)])
    pltpu.sync_copy(t_hbm.at[pl.ds(base, BPW)], tidx_v.at[pl.ds(0, BPW)])

    row_bufs = ((e1h0, e2h0, r10, r20, e1t0, e2t0),
                (e1h1, e2h1, r11, r21, e1t1, e2t1))
    sems = (sem0, sem1)

    def load_idx(jj):
        isl = pl.ds(jj * C, L)
        return hidx_v[isl], ridx_v[isl], tidx_v[isl]

    def fire(jj, b):
        e1h, e2h, r1v, r2v, e1t, e2t = row_bufs[b]
        sem = sems[b]
        hid, rid, tid = load_idx(jj)
        for k in range(C):
            hrow = pl.multiple_of((hid[k] >> 3) * 8, 8)
            rrow = pl.multiple_of((rid[k] >> 3) * 8, 8)
            trow = pl.multiple_of((tid[k] >> 3) * 8, 8)
            pltpu.async_copy(e1_hbm.at[pl.ds(hrow, 8), :], e1h.at[k], sem)
            pltpu.async_copy(e2_hbm.at[pl.ds(hrow, 8), :], e2h.at[k], sem)
            pltpu.async_copy(r1_hbm.at[pl.ds(rrow, 8), :], r1v.at[k], sem)
            pltpu.async_copy(r2_hbm.at[pl.ds(rrow, 8), :], r2v.at[k], sem)
            pltpu.async_copy(e1_hbm.at[pl.ds(trow, 8), :], e1t.at[k], sem)
            pltpu.async_copy(e2_hbm.at[pl.ds(trow, 8), :], e2t.at[k], sem)

    def drain_and_compute(jj, b, res):
        e1h, e2h, r1v, r2v, e1t, e2t = row_bufs[b]
        sem = sems[b]
        for buf in (e1h, e2h, r1v, r2v, e1t, e2t):
            pltpu.make_async_copy(
                e1_hbm.at[pl.ds(0, 8 * C), :], buf, sem).wait()
        hid, rid, tid = load_idx(jj)
        for k in range(C):
            hs = hid[k] & 7
            rs = rid[k] & 7
            ts = tid[k] & 7
            acc = jnp.zeros((L,), jnp.float32)
            for s in range(NSL):
                sl = pl.ds(s * L, L)
                acc = (acc
                       + e1h[k, hs, sl] * r1v[k, rs, sl] * e2t[k, ts, sl]
                       + e2h[k, hs, sl] * r2v[k, rs, sl] * e1t[k, ts, sl])
            res = jnp.where(lane == b * C + k, jnp.sum(acc), res)
        return res

    fire(0, 0)

    def body(j2, carry):
        res = jnp.zeros((L,), jnp.float32)
        for b in (0, 1):
            jj = 2 * j2 + b

            @pl.when(jj + 1 < NCH)
            def _():
                fire(jj + 1, 1 - b)

            res = drain_and_compute(jj, b, res)
        out_v[...] = res * 0.5
        pltpu.sync_copy(out_v, out_hbm.at[pl.ds(base + j2 * L, L)])
        return carry

    lax.fori_loop(0, NCH // 2, body, 0)


def kernel(h_idx, r_idx, t_idx, E1, E2, R1, R2):
    h = h_idx.astype(jnp.int32)
    r = r_idx.astype(jnp.int32)
    t = t_idx.astype(jnp.int32)
    mesh = plsc.VectorSubcoreMesh(core_axis_name="c", subcore_axis_name="s")
    idx_t = pltpu.VMEM((BPW + L,), jnp.int32)
    row_t = pltpu.VMEM((C, 8, D), jnp.float32)
    fn = pl.kernel(
        _sc_body,
        mesh=mesh,
        compiler_params=pltpu.CompilerParams(needs_layout_passes=False),
        out_type=jax.ShapeDtypeStruct((B,), jnp.float32),
        scratch_types=(
            [idx_t] * 3 + [row_t] * 12
            + [pltpu.VMEM((L,), jnp.float32),
               pltpu.SemaphoreType.DMA, pltpu.SemaphoreType.DMA]
        ),
    )
    return fn(h, r, t, E1, E2, R1, R2)


# speedup vs baseline: 1.4054x; 1.4054x over previous
import jax
import jax.numpy as jnp
from jax import lax
from jax.experimental import pallas as pl
from jax.experimental.pallas import tpu as pltpu
from jax.experimental.pallas import tpu_sc as plsc

NC = 2
NS = 16
NW = NC * NS
L = 16
B = 16384
D = 64
BPW = B // NW
C = 8
NCH = BPW // C
NSL = D // L


def _sc_body(h_hbm, r_hbm, t_hbm, e1_hbm, e2_hbm, r1_hbm, r2_hbm, out_hbm,
             hidx_v, ridx_v, tidx_v,
             e1h0, e2h0, r10, r20, e1t0, e2t0,
             e1h1, e2h1, r11, r21, e1t1, e2t1,
             out_v, sem0, sem1):
    cid = lax.axis_index("c")
    sid = lax.axis_index("s")
    wid = sid * NC + cid
    base = wid * BPW
    lane = lax.iota(jnp.int32, L)

    pltpu.sync_copy(h_hbm.at[pl.ds(base, BPW)], hidx_v.at[pl.ds(0, BPW)])
    pltpu.sync_copy(r_hbm.at[pl.ds(base, BPW)], ridx_v.at[pl.ds(0, BPW)])
    pltpu.sync_copy(t_hbm.at[pl.ds(base, BPW)], tidx_v.at[pl.ds(0, BPW)])

    row_bufs = ((e1h0, e2h0, r10, r20, e1t0, e2t0),
                (e1h1, e2h1, r11, r21, e1t1, e2t1))
    sems = (sem0, sem1)

    def load_idx(jj):
        isl = pl.ds(jj * C, L)
        return hidx_v[isl], ridx_v[isl], tidx_v[isl]

    def fire(jj, b):
        e1h, e2h, r1v, r2v, e1t, e2t = row_bufs[b]
        sem = sems[b]
        hid, rid, tid = load_idx(jj)
        for k in range(C):
            hrow = hid[k] >> 3
            rrow = rid[k] >> 3
            trow = tid[k] >> 3
            pltpu.async_copy(e1_hbm.at[hrow], e1h.at[k], sem)
            pltpu.async_copy(e2_hbm.at[hrow], e2h.at[k], sem)
            pltpu.async_copy(r1_hbm.at[rrow], r1v.at[k], sem)
            pltpu.async_copy(r2_hbm.at[rrow], r2v.at[k], sem)
            pltpu.async_copy(e1_hbm.at[trow], e1t.at[k], sem)
            pltpu.async_copy(e2_hbm.at[trow], e2t.at[k], sem)

    def drain_and_compute(jj, b, res):
        e1h, e2h, r1v, r2v, e1t, e2t = row_bufs[b]
        sem = sems[b]
        for buf in (e1h, e2h, r1v, r2v, e1t, e2t):
            pltpu.make_async_copy(
                e1_hbm.at[pl.ds(0, C)], buf, sem).wait()
        hid, rid, tid = load_idx(jj)
        for k in range(C):
            hs = hid[k] & 7
            rs = rid[k] & 7
            ts = tid[k] & 7
            acc = jnp.zeros((L,), jnp.float32)
            for s in range(NSL):
                sl = pl.ds(s * L, L)
                acc = (acc
                       + e1h[k, hs, sl] * r1v[k, rs, sl] * e2t[k, ts, sl]
                       + e2h[k, hs, sl] * r2v[k, rs, sl] * e1t[k, ts, sl])
            res = jnp.where(lane == b * C + k, jnp.sum(acc), res)
        return res

    fire(0, 0)

    def body(j2, carry):
        res = jnp.zeros((L,), jnp.float32)
        for b in (0, 1):
            jj = 2 * j2 + b

            @pl.when(jj + 1 < NCH)
            def _():
                fire(jj + 1, 1 - b)

            res = drain_and_compute(jj, b, res)
        out_v[...] = res * 0.5
        pltpu.sync_copy(out_v, out_hbm.at[pl.ds(base + j2 * L, L)])
        return carry

    lax.fori_loop(0, NCH // 2, body, 0)


def kernel(h_idx, r_idx, t_idx, E1, E2, R1, R2):
    h = h_idx.astype(jnp.int32)
    r = r_idx.astype(jnp.int32)
    t = t_idx.astype(jnp.int32)
    mesh = plsc.VectorSubcoreMesh(core_axis_name="c", subcore_axis_name="s")
    idx_t = pltpu.VMEM((BPW + L,), jnp.int32)
    row_t = pltpu.VMEM((C, 8, D), jnp.float32)
    fn = pl.kernel(
        _sc_body,
        mesh=mesh,
        compiler_params=pltpu.CompilerParams(needs_layout_passes=False),
        out_type=jax.ShapeDtypeStruct((B,), jnp.float32),
        scratch_types=(
            [idx_t] * 3 + [row_t] * 12
            + [pltpu.VMEM((L,), jnp.float32),
               pltpu.SemaphoreType.DMA, pltpu.SemaphoreType.DMA]
        ),
    )
    return fn(h, r, t,
              E1.reshape(-1, 8, D), E2.reshape(-1, 8, D),
              R1.reshape(-1, 8, D), R2.reshape(-1, 8, D))


# speedup vs baseline: 1.4181x; 1.0090x over previous
import jax
import jax.numpy as jnp
from jax import lax
from jax.experimental import pallas as pl
from jax.experimental.pallas import tpu as pltpu
from jax.experimental.pallas import tpu_sc as plsc

NC = 2
NS = 16
NW = NC * NS
L = 16
B = 16384
D = 64
BPW = B // NW
C = 4
NB = 4
NCH = BPW // C
NSL = D // L


def _sc_body(h_hbm, r_hbm, t_hbm, e1_hbm, e2_hbm, r1_hbm, r2_hbm, out_hbm,
             hidx_v, ridx_v, tidx_v, *rest):
    row_bufs = tuple(tuple(rest[6 * b:6 * b + 6]) for b in range(NB))
    out_v = rest[6 * NB]
    sems = tuple(rest[6 * NB + 1:6 * NB + 1 + NB])
    cid = lax.axis_index("c")
    sid = lax.axis_index("s")
    wid = sid * NC + cid
    base = wid * BPW
    lane = lax.iota(jnp.int32, L)

    pltpu.sync_copy(h_hbm.at[pl.ds(base, BPW)], hidx_v.at[pl.ds(0, BPW)])
    pltpu.sync_copy(r_hbm.at[pl.ds(base, BPW)], ridx_v.at[pl.ds(0, BPW)])
    pltpu.sync_copy(t_hbm.at[pl.ds(base, BPW)], tidx_v.at[pl.ds(0, BPW)])

    def load_idx(jj):
        isl = pl.ds(jj * C, L)
        return hidx_v[isl], ridx_v[isl], tidx_v[isl]

    def fire(jj, b):
        e1h, e2h, r1v, r2v, e1t, e2t = row_bufs[b]
        sem = sems[b]
        hid, rid, tid = load_idx(jj)
        for k in range(C):
            hrow = hid[k] >> 3
            rrow = rid[k] >> 3
            trow = tid[k] >> 3
            pltpu.async_copy(e1_hbm.at[hrow], e1h.at[k], sem)
            pltpu.async_copy(e2_hbm.at[hrow], e2h.at[k], sem)
            pltpu.async_copy(r1_hbm.at[rrow], r1v.at[k], sem)
            pltpu.async_copy(r2_hbm.at[rrow], r2v.at[k], sem)
            pltpu.async_copy(e1_hbm.at[trow], e1t.at[k], sem)
            pltpu.async_copy(e2_hbm.at[trow], e2t.at[k], sem)

    def drain_and_compute(jj, b, res):
        e1h, e2h, r1v, r2v, e1t, e2t = row_bufs[b]
        sem = sems[b]
        for buf in (e1h, e2h, r1v, r2v, e1t, e2t):
            pltpu.make_async_copy(
                e1_hbm.at[pl.ds(0, C)], buf, sem).wait()
        hid, rid, tid = load_idx(jj)
        for k in range(C):
            hs = hid[k] & 7
            rs = rid[k] & 7
            ts = tid[k] & 7
            acc = jnp.zeros((L,), jnp.float32)
            for s in range(NSL):
                sl = pl.ds(s * L, L)
                acc = (acc
                       + e1h[k, hs, sl] * r1v[k, rs, sl] * e2t[k, ts, sl]
                       + e2h[k, hs, sl] * r2v[k, rs, sl] * e1t[k, ts, sl])
            res = jnp.where(lane == b * C + k, jnp.sum(acc), res)
        return res

    for p in range(NB - 1):
        fire(p, p)

    def body(j4, carry):
        res = jnp.zeros((L,), jnp.float32)
        for b in range(NB):
            jj = NB * j4 + b

            @pl.when(jj + NB - 1 < NCH)
            def _():
                fire(jj + NB - 1, (b + NB - 1) % NB)

            res = drain_and_compute(jj, b, res)
        out_v[...] = res * 0.5
        pltpu.sync_copy(out_v, out_hbm.at[pl.ds(base + j4 * L, L)])
        return carry

    lax.fori_loop(0, NCH // NB, body, 0)


def kernel(h_idx, r_idx, t_idx, E1, E2, R1, R2):
    h = h_idx.astype(jnp.int32)
    r = r_idx.astype(jnp.int32)
    t = t_idx.astype(jnp.int32)
    mesh = plsc.VectorSubcoreMesh(core_axis_name="c", subcore_axis_name="s")
    idx_t = pltpu.VMEM((BPW + L,), jnp.int32)
    row_t = pltpu.VMEM((C, 8, D), jnp.float32)
    fn = pl.kernel(
        _sc_body,
        mesh=mesh,
        compiler_params=pltpu.CompilerParams(needs_layout_passes=False),
        out_type=jax.ShapeDtypeStruct((B,), jnp.float32),
        scratch_types=(
            [idx_t] * 3 + [row_t] * (6 * NB)
            + [pltpu.VMEM((L,), jnp.float32)]
            + [pltpu.SemaphoreType.DMA] * NB
        ),
    )
    return fn(h, r, t,
              E1.reshape(-1, 8, D), E2.reshape(-1, 8, D),
              R1.reshape(-1, 8, D), R2.reshape(-1, 8, D))


# speedup vs baseline: 1.4224x; 1.0030x over previous
import jax
import jax.numpy as jnp
from jax import lax
from jax.experimental import pallas as pl
from jax.experimental.pallas import tpu as pltpu
from jax.experimental.pallas import tpu_sc as plsc

NC = 2
NS = 16
NW = NC * NS
L = 16
B = 16384
D = 64
BPW = B // NW
C = 4
NB = 4
NCH = BPW // C
NSL = D // L


def _sc_body(h_hbm, r_hbm, t_hbm, e1_hbm, e2_hbm, r1_hbm, r2_hbm, out_hbm,
             hidx_v, ridx_v, tidx_v, *rest):
    row_bufs = tuple(rest[:NB])
    out_v = rest[NB]
    sems = tuple(rest[NB + 1:NB + 1 + NB])
    cid = lax.axis_index("c")
    sid = lax.axis_index("s")
    wid = sid * NC + cid
    base = wid * BPW
    lane = lax.iota(jnp.int32, L)

    pltpu.sync_copy(h_hbm.at[pl.ds(base, BPW)], hidx_v.at[pl.ds(0, BPW)])
    pltpu.sync_copy(r_hbm.at[pl.ds(base, BPW)], ridx_v.at[pl.ds(0, BPW)])
    pltpu.sync_copy(t_hbm.at[pl.ds(base, BPW)], tidx_v.at[pl.ds(0, BPW)])

    def load_idx(jj):
        isl = pl.ds(jj * C, L)
        return hidx_v[isl], ridx_v[isl], tidx_v[isl]

    def fire(jj, b):
        rows = row_bufs[b]
        sem = sems[b]
        hid, rid, tid = load_idx(jj)
        for k in range(C):
            hrow = hid[k] >> 3
            rrow = rid[k] >> 3
            trow = tid[k] >> 3
            pltpu.async_copy(e1_hbm.at[hrow], rows.at[0, k], sem)
            pltpu.async_copy(e2_hbm.at[hrow], rows.at[1, k], sem)
            pltpu.async_copy(r1_hbm.at[rrow], rows.at[2, k], sem)
            pltpu.async_copy(r2_hbm.at[rrow], rows.at[3, k], sem)
            pltpu.async_copy(e1_hbm.at[trow], rows.at[4, k], sem)
            pltpu.async_copy(e2_hbm.at[trow], rows.at[5, k], sem)

    def drain_and_compute(jj, b, res):
        rows = row_bufs[b]
        sem = sems[b]
        pltpu.make_async_copy(
            e1_hbm.at[pl.ds(0, 6 * C)], rows, sem).wait()
        hid, rid, tid = load_idx(jj)
        for k in range(C):
            hs = hid[k] & 7
            rs = rid[k] & 7
            ts = tid[k] & 7
            acc = jnp.zeros((L,), jnp.float32)
            for s in range(NSL):
                sl = pl.ds(s * L, L)
                acc = (acc
                       + rows[0, k, hs, sl] * rows[2, k, rs, sl]
                       * rows[5, k, ts, sl]
                       + rows[1, k, hs, sl] * rows[3, k, rs, sl]
                       * rows[4, k, ts, sl])
            res = jnp.where(lane == b * C + k, jnp.sum(acc), res)
        return res

    for p in range(NB - 1):
        fire(p, p)

    def body(j4, carry):
        res = jnp.zeros((L,), jnp.float32)
        for b in range(NB):
            jj = NB * j4 + b

            @pl.when(jj + NB - 1 < NCH)
            def _():
                fire(jj + NB - 1, (b + NB - 1) % NB)

            res = drain_and_compute(jj, b, res)
        out_v[...] = res * 0.5
        pltpu.sync_copy(out_v, out_hbm.at[pl.ds(base + j4 * L, L)])
        return carry

    lax.fori_loop(0, NCH // NB, body, 0)


def kernel(h_idx, r_idx, t_idx, E1, E2, R1, R2):
    h = h_idx.astype(jnp.int32)
    r = r_idx.astype(jnp.int32)
    t = t_idx.astype(jnp.int32)
    mesh = plsc.VectorSubcoreMesh(core_axis_name="c", subcore_axis_name="s")
    idx_t = pltpu.VMEM((BPW + L,), jnp.int32)
    row_t = pltpu.VMEM((6, C, 8, D), jnp.float32)
    fn = pl.kernel(
        _sc_body,
        mesh=mesh,
        compiler_params=pltpu.CompilerParams(needs_layout_passes=False),
        out_type=jax.ShapeDtypeStruct((B,), jnp.float32),
        scratch_types=(
            [idx_t] * 3 + [row_t] * NB
            + [pltpu.VMEM((L,), jnp.float32)]
            + [pltpu.SemaphoreType.DMA] * NB
        ),
    )
    return fn(h, r, t,
              E1.reshape(-1, 8, D), E2.reshape(-1, 8, D),
              R1.reshape(-1, 8, D), R2.reshape(-1, 8, D))
